# speedup vs baseline: 3.0996x; 3.0996x over previous
import jax
import jax.numpy as jnp
from jax.experimental import pallas as pl
from jax.experimental.pallas import tpu as pltpu

B, L, D = 4, 8192, 768
K = 4096
CHUNK = 512
NCHUNK = L // CHUNK


def _pa_kernel(x_ref, out_ref, s2_ref):
    i = pl.program_id(0)
    x = x_ref[...]
    s2_ref[:, pl.ds(i * CHUNK, CHUNK)] = jnp.sum(x * x, axis=2)

    @pl.when(i == NCHUNK - 1)
    def _finalize():
        sigma = jnp.sqrt(s2_ref[...])
        smax = jnp.max(sigma)
        a = jnp.tanh(1.0 - sigma / smax)
        shifted = a - jnp.max(a, axis=1, keepdims=True)
        logsm = shifted - jnp.log(jnp.sum(jnp.exp(shifted), axis=1, keepdims=True))
        att = jnp.exp(logsm)

        v = jax.lax.bitcast_convert_type(att, jnp.int32)

        def body_val(_, lohi):
            lo, hi = lohi
            mid = lo + (hi - lo) // 2
            cnt = jnp.sum((v <= mid).astype(jnp.int32), axis=1, keepdims=True)
            ge = cnt >= K
            return jnp.where(ge, lo, mid + 1), jnp.where(ge, mid, hi)

        lo0 = jnp.zeros((B, 1), jnp.int32)
        hi0 = jnp.full((B, 1), 0x3F800000, jnp.int32)
        _, t = jax.lax.fori_loop(0, 31, body_val, (lo0, hi0))

        c_less = jnp.sum((v < t).astype(jnp.int32), axis=1, keepdims=True)
        m = K - c_less
        eq = v == t
        idx = jax.lax.broadcasted_iota(jnp.int32, (B, L), 1)

        def body_idx(_, lohi):
            lo, hi = lohi
            mid = lo + (hi - lo) // 2
            cnt = jnp.sum((eq & (idx <= mid)).astype(jnp.int32), axis=1,
                          keepdims=True)
            ge = cnt >= m
            return jnp.where(ge, lo, mid + 1), jnp.where(ge, mid, hi)

        li0 = jnp.zeros((B, 1), jnp.int32)
        hi1 = jnp.full((B, 1), L - 1, jnp.int32)
        _, j = jax.lax.fori_loop(0, 13, body_idx, (li0, hi1))

        zero = (v < t) | (eq & (idx <= j))
        out_ref[...] = jnp.where(zero, 0.0, att)


def kernel(delta):
    out = pl.pallas_call(
        _pa_kernel,
        grid=(NCHUNK,),
        in_specs=[pl.BlockSpec((B, CHUNK, D), lambda i: (0, i, 0))],
        out_specs=pl.BlockSpec((B, L), lambda i: (0, 0)),
        out_shape=jax.ShapeDtypeStruct((B, L), jnp.float32),
        scratch_shapes=[pltpu.VMEM((B, L), jnp.float32)],
        compiler_params=pltpu.CompilerParams(
            dimension_semantics=("arbitrary",),
        ),
    )(delta)
    return out[..., None]
